# baseline (speedup 1.0000x reference)
import functools

import jax
import jax.numpy as jnp
from jax import lax
from jax.experimental import pallas as pl
from jax.experimental.pallas import tpu as pltpu
from jax.experimental.pallas import tpu_sc as plsc

N = 10000
E = 320000
DX = 128
DE = 16

NC = 2
NS = 16
NW = NC * NS

CHUNK = 80
EDGES_PER_TILE = E // NW
CHUNKS_PER_TILE = EDGES_PER_TILE // CHUNK
STAGE_CHUNKS = 25
STAGE_EDGES = STAGE_CHUNKS * CHUNK
NUM_STAGES = CHUNKS_PER_TILE // STAGE_CHUNKS
N_PAD = 10240
ROWS_PER_TILE = N_PAD // NS


def _sc_segment_sum(idx1d, edge_attr):
    mesh = plsc.VectorSubcoreMesh(core_axis_name="c", subcore_axis_name="s")

    @functools.partial(
        pl.kernel,
        mesh=mesh,
        out_type=jax.ShapeDtypeStruct((NC, N_PAD, DE), jnp.float32),
        scratch_types=[
            pltpu.VMEM((EDGES_PER_TILE,), jnp.int32),
            pltpu.VMEM((STAGE_EDGES, DE), jnp.float32),
            pltpu.VMEM((STAGE_EDGES, DE), jnp.float32),
            pltpu.VMEM_SHARED((N_PAD, DE), jnp.float32),
            pltpu.SemaphoreType.DMA,
            pltpu.SemaphoreType.DMA,
        ],
        compiler_params=pltpu.CompilerParams(use_tc_tiling_on_sc=False),
    )
    def sc_kernel(idx_hbm, attr_hbm, out_hbm, idx_v, stage0, stage1, agg,
                  sem0, sem1):
        cid = lax.axis_index("c")
        sid = lax.axis_index("s")
        wid = cid * NS + sid
        base_edge = wid * EDGES_PER_TILE

        zvec = jnp.zeros((DE,), jnp.float32)

        def zbody(i, carry):
            stage0[i, :] = zvec
            return carry

        lax.fori_loop(0, ROWS_PER_TILE, zbody, 0)
        pltpu.sync_copy(stage0.at[pl.ds(0, ROWS_PER_TILE)],
                        agg.at[pl.ds(sid * ROWS_PER_TILE, ROWS_PER_TILE)])

        pltpu.sync_copy(idx_hbm.at[pl.ds(base_edge, EDGES_PER_TILE)], idx_v)
        plsc.subcore_barrier()

        stages = (stage0, stage1)
        sems = (sem0, sem1)

        def start(s):
            b = s % 2
            return pltpu.async_copy(
                attr_hbm.at[pl.ds(base_edge + s * STAGE_EDGES, STAGE_EDGES)],
                stages[b], sems[b])

        cps = {0: start(0)}
        for s in range(NUM_STAGES):
            if s + 1 < NUM_STAGES:
                cps[(s + 1) % 2] = start(s + 1)
            cps[s % 2].wait()
            stg = stages[s % 2]

            def scat(k, carry, stg=stg, s=s):
                pltpu.sync_copy(
                    stg.at[pl.ds(k * CHUNK, CHUNK)],
                    agg.at[idx_v.at[pl.ds((s * STAGE_CHUNKS + k) * CHUNK, CHUNK)]],
                    add=True)
                return carry

            lax.fori_loop(0, STAGE_CHUNKS, scat, 0)

        plsc.subcore_barrier()
        pltpu.sync_copy(
            agg.at[pl.ds(sid * ROWS_PER_TILE, ROWS_PER_TILE)],
            out_hbm.at[cid, pl.ds(sid * ROWS_PER_TILE, ROWS_PER_TILE)])

    return sc_kernel(idx1d, edge_attr)


def _tc_linear(x, partials, Wx, bx2, We, be2):
    R = 2000

    def body(x_ref, p_ref, wx_ref, bx_ref, we_ref, be_ref, o_ref):
        hx = lax.dot_general(x_ref[...], wx_ref[...],
                             (((1,), (1,)), ((), ())),
                             preferred_element_type=jnp.float32)
        hx = hx + bx_ref[...]
        aggb = p_ref[0] + p_ref[1]
        he = lax.dot_general(aggb, we_ref[...],
                             (((1,), (1,)), ((), ())),
                             preferred_element_type=jnp.float32)
        he = he + be_ref[...]
        o_ref[:, :DX] = jnp.maximum(hx, 0.0)
        o_ref[:, DX:] = jnp.maximum(he, 0.0)

    return pl.pallas_call(
        body,
        grid=(N // R,),
        in_specs=[
            pl.BlockSpec((R, DX), lambda i: (i, 0)),
            pl.BlockSpec((NC, R, DE), lambda i: (0, i, 0)),
            pl.BlockSpec((DX, DX), lambda i: (0, 0)),
            pl.BlockSpec((1, DX), lambda i: (0, 0)),
            pl.BlockSpec((DE, DE), lambda i: (0, 0)),
            pl.BlockSpec((1, DE), lambda i: (0, 0)),
        ],
        out_specs=pl.BlockSpec((R, DX + DE), lambda i: (i, 0)),
        out_shape=jax.ShapeDtypeStruct((N, DX + DE), jnp.float32),
    )(x, partials, Wx, bx2, We, be2)


def kernel(x, edge_index, edge_attr, Wx, bx, We, be):
    idx1d = edge_index[0].astype(jnp.int32)
    partials = _sc_segment_sum(idx1d, edge_attr)
    return _tc_linear(x, partials, Wx, bx.reshape(1, DX), We, be.reshape(1, DE))


# speedup vs baseline: 1.4759x; 1.4759x over previous
import functools

import jax
import jax.numpy as jnp
from jax import lax
from jax.experimental import pallas as pl
from jax.experimental.pallas import tpu as pltpu
from jax.experimental.pallas import tpu_sc as plsc

N = 10000
E = 320000
DX = 128
DE = 16

NC = 2
NS = 16

HALF = E // NC
CH = 16000
NPASS = HALF // CH
GROUPS = CH // 16
N_PAD = 10240
ZGROUPS = N_PAD // 16


def _sc_segment_sum_t(idx1d, attr_t):
    mesh = plsc.VectorSubcoreMesh(core_axis_name="c", subcore_axis_name="s")

    @functools.partial(
        pl.kernel,
        mesh=mesh,
        out_type=jax.ShapeDtypeStruct((NC, DE, N_PAD), jnp.float32),
        scratch_types=[
            pltpu.VMEM((CH,), jnp.int32),
            pltpu.VMEM((CH,), jnp.int32),
            pltpu.VMEM((CH,), jnp.float32),
            pltpu.VMEM((CH,), jnp.float32),
            pltpu.VMEM((N_PAD,), jnp.float32),
            pltpu.SemaphoreType.DMA,
            pltpu.SemaphoreType.DMA,
        ],
        compiler_params=pltpu.CompilerParams(use_tc_tiling_on_sc=False,
                                             needs_layout_passes=False),
    )
    def sc_kernel(idx_hbm, attr_hbm, out_hbm, idx0, idx1, val0, val1, plane,
                  sem0, sem1):
        cid = lax.axis_index("c")
        f = lax.axis_index("s")
        ebase = cid * HALF

        zvec = jnp.zeros((16,), jnp.float32)

        def zbody(i, carry):
            plane[pl.ds(16 * i, 16)] = zvec
            return carry

        lax.fori_loop(0, ZGROUPS, zbody, 0)

        idxb = (idx0, idx1)
        valb = (val0, val1)
        sems = (sem0, sem1)

        def start(p):
            b = p % 2
            ci = pltpu.async_copy(
                idx_hbm.at[pl.ds(ebase + p * CH, CH)], idxb[b], sems[b])
            cv = pltpu.async_copy(
                attr_hbm.at[f, pl.ds(ebase + p * CH, CH)], valb[b], sems[b])
            return ci, cv

        cps = {0: start(0)}
        for p in range(NPASS):
            b = p % 2
            if p + 1 < NPASS:
                cps[(p + 1) % 2] = start(p + 1)
            ci, cv = cps[b]
            ci.wait()
            cv.wait()
            ib, vb = idxb[b], valb[b]

            def inner(i, carry, ib=ib, vb=vb):
                idxv = ib[pl.ds(16 * i, 16)]
                vals = vb[pl.ds(16 * i, 16)]
                plsc.addupdate_scatter(plane, [idxv], vals)
                return carry

            lax.fori_loop(0, GROUPS, inner, 0)

        pltpu.sync_copy(plane, out_hbm.at[cid, f])

    return sc_kernel(idx1d, attr_t)


def _tc_linear_t(x, partials, Wx, bxc, We, bec):
    L = 2048
    grid = (pl.cdiv(N, L),)

    def body(x_ref, p_ref, wx_ref, bx_ref, we_ref, be_ref, o_ref):
        hx = lax.dot_general(wx_ref[...], x_ref[...],
                             (((1,), (1,)), ((), ())),
                             preferred_element_type=jnp.float32)
        hx = hx + bx_ref[...]
        aggt = p_ref[0] + p_ref[1]
        he = lax.dot_general(we_ref[...], aggt,
                             (((1,), (0,)), ((), ())),
                             preferred_element_type=jnp.float32)
        he = he + be_ref[...]
        o_ref[:DX, :] = jnp.maximum(hx, 0.0)
        o_ref[DX:, :] = jnp.maximum(he, 0.0)

    return pl.pallas_call(
        body,
        grid=grid,
        in_specs=[
            pl.BlockSpec((L, DX), lambda i: (i, 0)),
            pl.BlockSpec((NC, DE, L), lambda i: (0, 0, i)),
            pl.BlockSpec((DX, DX), lambda i: (0, 0)),
            pl.BlockSpec((DX, 1), lambda i: (0, 0)),
            pl.BlockSpec((DE, DE), lambda i: (0, 0)),
            pl.BlockSpec((DE, 1), lambda i: (0, 0)),
        ],
        out_specs=pl.BlockSpec((DX + DE, L), lambda i: (0, i)),
        out_shape=jax.ShapeDtypeStruct((DX + DE, N), jnp.float32),
    )(x, partials, Wx, bxc, We, bec)


def kernel(x, edge_index, edge_attr, Wx, bx, We, be):
    idx1d = edge_index[0].astype(jnp.int32)
    partials = _sc_segment_sum_t(idx1d, edge_attr.T)
    out_t = _tc_linear_t(x, partials, Wx, bx.reshape(DX, 1), We,
                         be.reshape(DE, 1))
    return out_t.T


# speedup vs baseline: 1.5044x; 1.0193x over previous
import functools

import jax
import jax.numpy as jnp
from jax import lax
from jax.experimental import pallas as pl
from jax.experimental.pallas import tpu as pltpu
from jax.experimental.pallas import tpu_sc as plsc

N = 10000
E = 320000
DX = 128
DE = 16

NC = 2
NS = 16

HALF = E // NC
CH = 16000
NPASS = HALF // CH
GROUPS = CH // 16
N_PAD = 10240
ZGROUPS = N_PAD // 16


def _sc_segment_sum_t(idx1d, attr_t):
    mesh = plsc.VectorSubcoreMesh(core_axis_name="c", subcore_axis_name="s")

    @functools.partial(
        pl.kernel,
        mesh=mesh,
        out_type=jax.ShapeDtypeStruct((NC, DE, N_PAD), jnp.float32),
        scratch_types=[
            pltpu.VMEM((CH,), jnp.int32),
            pltpu.VMEM((CH,), jnp.int32),
            pltpu.VMEM((CH,), jnp.float32),
            pltpu.VMEM((CH,), jnp.float32),
            pltpu.VMEM((N_PAD,), jnp.float32),
            pltpu.SemaphoreType.DMA,
            pltpu.SemaphoreType.DMA,
        ],
        compiler_params=pltpu.CompilerParams(use_tc_tiling_on_sc=False,
                                             needs_layout_passes=False),
    )
    def sc_kernel(idx_hbm, attr_hbm, out_hbm, idx0, idx1, val0, val1, plane,
                  sem0, sem1):
        cid = lax.axis_index("c")
        f = lax.axis_index("s")
        ebase = cid * HALF

        zvec = jnp.zeros((16,), jnp.float32)

        def zbody(i, carry):
            plane[pl.ds(16 * i, 16)] = zvec
            return carry

        lax.fori_loop(0, ZGROUPS, zbody, 0)

        idxb = (idx0, idx1)
        valb = (val0, val1)
        sems = (sem0, sem1)

        def start(p):
            b = p % 2
            ci = pltpu.async_copy(
                idx_hbm.at[pl.ds(ebase + p * CH, CH)], idxb[b], sems[b])
            cv = pltpu.async_copy(
                attr_hbm.at[f, pl.ds(ebase + p * CH, CH)], valb[b], sems[b])
            return ci, cv

        cps = {0: start(0)}
        for p in range(NPASS):
            b = p % 2
            if p + 1 < NPASS:
                cps[(p + 1) % 2] = start(p + 1)
            ci, cv = cps[b]
            ci.wait()
            cv.wait()
            ib, vb = idxb[b], valb[b]

            def inner(i, carry, ib=ib, vb=vb):
                base = 128 * i
                for u in range(8):
                    idxv = ib[pl.ds(base + 16 * u, 16)]
                    vals = vb[pl.ds(base + 16 * u, 16)]
                    plsc.addupdate_scatter(plane, [idxv], vals)
                return carry

            lax.fori_loop(0, GROUPS // 8, inner, 0)

        pltpu.sync_copy(plane, out_hbm.at[cid, f])

    return sc_kernel(idx1d, attr_t)


def _tc_linear_t(x, partials, Wx, bxc, We, bec):
    L = 2048
    grid = (pl.cdiv(N, L),)

    def body(x_ref, p_ref, wx_ref, bx_ref, we_ref, be_ref, o_ref):
        hx = lax.dot_general(wx_ref[...], x_ref[...],
                             (((1,), (1,)), ((), ())),
                             preferred_element_type=jnp.float32)
        hx = hx + bx_ref[...]
        aggt = p_ref[0] + p_ref[1]
        he = lax.dot_general(we_ref[...], aggt,
                             (((1,), (0,)), ((), ())),
                             preferred_element_type=jnp.float32)
        he = he + be_ref[...]
        o_ref[:DX, :] = jnp.maximum(hx, 0.0)
        o_ref[DX:, :] = jnp.maximum(he, 0.0)

    return pl.pallas_call(
        body,
        grid=grid,
        in_specs=[
            pl.BlockSpec((L, DX), lambda i: (i, 0)),
            pl.BlockSpec((NC, DE, L), lambda i: (0, 0, i)),
            pl.BlockSpec((DX, DX), lambda i: (0, 0)),
            pl.BlockSpec((DX, 1), lambda i: (0, 0)),
            pl.BlockSpec((DE, DE), lambda i: (0, 0)),
            pl.BlockSpec((DE, 1), lambda i: (0, 0)),
        ],
        out_specs=pl.BlockSpec((DX + DE, L), lambda i: (0, i)),
        out_shape=jax.ShapeDtypeStruct((DX + DE, N), jnp.float32),
    )(x, partials, Wx, bxc, We, bec)


def kernel(x, edge_index, edge_attr, Wx, bx, We, be):
    idx1d = edge_index[0].astype(jnp.int32)
    partials = _sc_segment_sum_t(idx1d, edge_attr.T)
    out_t = _tc_linear_t(x, partials, Wx, bx.reshape(DX, 1), We,
                         be.reshape(DE, 1))
    return out_t.T


# speedup vs baseline: 2.0375x; 1.3544x over previous
import functools

import jax
import jax.numpy as jnp
from jax import lax
from jax.experimental import pallas as pl
from jax.experimental.pallas import tpu as pltpu
from jax.experimental.pallas import tpu_sc as plsc

N = 10000
E = 320000
DX = 128
DE = 16

NC = 2
NS = 16

HALF = E // NC
CH = 16000
NPASS = HALF // CH
GROUPS = CH // 16
N_PAD = 10240
ZGROUPS = N_PAD // 16


def _sc_segment_sum_t(idx1d, attr_t):
    mesh = plsc.VectorSubcoreMesh(core_axis_name="c", subcore_axis_name="s")

    @functools.partial(
        pl.kernel,
        mesh=mesh,
        out_type=jax.ShapeDtypeStruct((NC, DE, N_PAD), jnp.float32),
        scratch_types=[
            pltpu.VMEM((CH,), jnp.int32),
            pltpu.VMEM((CH,), jnp.int32),
            pltpu.VMEM((CH,), jnp.float32),
            pltpu.VMEM((CH,), jnp.float32),
            pltpu.VMEM((N_PAD,), jnp.float32),
            pltpu.SemaphoreType.DMA,
            pltpu.SemaphoreType.DMA,
        ],
        compiler_params=pltpu.CompilerParams(use_tc_tiling_on_sc=False,
                                             needs_layout_passes=False),
    )
    def sc_kernel(idx_hbm, attr_hbm, out_hbm, idx0, idx1, val0, val1, plane,
                  sem0, sem1):
        cid = lax.axis_index("c")
        f = lax.axis_index("s")
        ebase = cid * HALF

        zvec = jnp.zeros((16,), jnp.float32)

        def zbody(i, carry):
            plane[pl.ds(16 * i, 16)] = zvec
            return carry

        lax.fori_loop(0, ZGROUPS, zbody, 0)

        idxb = (idx0, idx1)
        valb = (val0, val1)
        sems = (sem0, sem1)

        def start(p):
            b = p % 2
            ci = pltpu.async_copy(
                idx_hbm.at[pl.ds(ebase + p * CH, CH)], idxb[b], sems[b])
            cv = pltpu.async_copy(
                attr_hbm.at[f, pl.ds(ebase + p * CH, CH)], valb[b], sems[b])
            return ci, cv

        cps = {0: start(0)}
        for p in range(NPASS):
            b = p % 2
            if p + 1 < NPASS:
                cps[(p + 1) % 2] = start(p + 1)
            ci, cv = cps[b]
            ci.wait()
            cv.wait()
            ib, vb = idxb[b], valb[b]

            @plsc.parallel_loop(0, GROUPS, unroll=8)
            def inner(i, ib=ib, vb=vb):
                idxv = ib[pl.ds(16 * i, 16)]
                vals = vb[pl.ds(16 * i, 16)]
                plsc.addupdate_scatter(plane, [idxv], vals)

        pltpu.sync_copy(plane, out_hbm.at[cid, f])

    return sc_kernel(idx1d, attr_t)


def _tc_linear_t(x, partials, Wx, bxc, We, bec):
    L = 2048
    grid = (pl.cdiv(N, L),)

    def body(x_ref, p_ref, wx_ref, bx_ref, we_ref, be_ref, o_ref):
        hx = lax.dot_general(wx_ref[...], x_ref[...],
                             (((1,), (1,)), ((), ())),
                             preferred_element_type=jnp.float32)
        hx = hx + bx_ref[...]
        aggt = p_ref[0] + p_ref[1]
        he = lax.dot_general(we_ref[...], aggt,
                             (((1,), (0,)), ((), ())),
                             preferred_element_type=jnp.float32)
        he = he + be_ref[...]
        o_ref[:DX, :] = jnp.maximum(hx, 0.0)
        o_ref[DX:, :] = jnp.maximum(he, 0.0)

    return pl.pallas_call(
        body,
        grid=grid,
        in_specs=[
            pl.BlockSpec((L, DX), lambda i: (i, 0)),
            pl.BlockSpec((NC, DE, L), lambda i: (0, 0, i)),
            pl.BlockSpec((DX, DX), lambda i: (0, 0)),
            pl.BlockSpec((DX, 1), lambda i: (0, 0)),
            pl.BlockSpec((DE, DE), lambda i: (0, 0)),
            pl.BlockSpec((DE, 1), lambda i: (0, 0)),
        ],
        out_specs=pl.BlockSpec((DX + DE, L), lambda i: (0, i)),
        out_shape=jax.ShapeDtypeStruct((DX + DE, N), jnp.float32),
    )(x, partials, Wx, bxc, We, bec)


def kernel(x, edge_index, edge_attr, Wx, bx, We, be):
    idx1d = edge_index[0].astype(jnp.int32)
    partials = _sc_segment_sum_t(idx1d, edge_attr.T)
    out_t = _tc_linear_t(x, partials, Wx, bx.reshape(DX, 1), We,
                         be.reshape(DE, 1))
    return out_t.T
